# Initial kernel scaffold; baseline (speedup 1.0000x reference)
import jax
import jax.numpy as jnp
from jax.experimental import pallas as pl


def kernel(edge_attr, params):
    raise NotImplementedError("write your pallas kernel here")


# speedup vs baseline: 29.7622x; 29.7622x over previous
import jax
import jax.numpy as jnp
from jax import lax
from jax.experimental import pallas as pl

_EMB_DIM = 128
_EDGE_CONT = {0: [3, 6, 7, 9, 10, 11, 12, 13], 1: [2, 3, 4, 5, 6, 7, 9, 10, 11, 12, 13], 2: [2, 3, 4, 5, 6, 7, 9, 10, 11, 12, 13], 3: [1, 4, 5, 6, 7, 9, 10, 11, 12, 13], 4: [2, 3, 4, 5, 6, 7, 9, 10, 11, 12, 13], 5: [1, 2, 3, 4, 5, 6, 7, 9, 10, 11, 12, 13], 6: [2, 3, 4, 5, 6, 7, 9, 10, 11, 12, 13], 7: [1, 2, 3, 4, 5, 6, 7, 9, 10, 11, 12, 13], 8: [0, 1, 4, 6, 7, 9, 10, 11, 12, 13]}
_EDGE_DISC_FEATS = {0: [0, 1, 2, 4, 5, 8], 1: [0, 1, 8], 2: [0, 1, 8], 3: [0, 2, 3, 8], 4: [0, 1, 8], 5: [0, 8], 6: [0, 1, 8], 7: [0, 8], 8: [2, 3, 5, 8]}
_COL_RANGES = {0: 4, 1: 6, 2: 6, 3: 8, 4: 15, 5: 2, 8: 9}
_OH_BASE = {}
_off = 0
for _c in sorted(_COL_RANGES):
    _OH_BASE[_c] = _off
    _off += _COL_RANGES[_c]
_OH_W = _off
_PHI_W = _OH_W + 14

_BLOCK = 3200


def _build_g(params):
    gs = []
    for t in range(9):
        g = jnp.zeros((_PHI_W, _EMB_DIM), jnp.float32)
        feats = _EDGE_DISC_FEATS[t]
        nd = len(feats)
        per, rem = _EMB_DIM // nd, _EMB_DIM % nd
        col = 0
        for i, f in enumerate(feats):
            dim = per + (1 if i < rem else 0)
            tbl = params["tables"][str(t)][str(f)]
            r = _COL_RANGES[f]
            g = g.at[_OH_BASE[f]:_OH_BASE[f] + r, col:col + dim].set(tbl[:r, :])
            col += dim
        g = g.at[_OH_BASE[8] + t, :].add(params["b"][str(t)])
        w = params["W"][str(t)]
        rows = jnp.array([_OH_W + c for c in _EDGE_CONT[t]], jnp.int32)
        g = g.at[rows, :].set(w.T)
        gs.append(g)
    return jnp.stack(gs)


def _body(a_ref, g_ref, o_ref):
    a = a_ref[:, :]
    b = a.shape[0]
    t = a[:, 8]
    i50 = lax.broadcasted_iota(jnp.int32, (b, _OH_W), 1)
    oh = jnp.zeros((b, _OH_W), jnp.float32)
    for c, base in _OH_BASE.items():
        tgt = base + a[:, c].astype(jnp.int32)
        oh = oh + (i50 == tgt[:, None]).astype(jnp.float32)
    phi = jnp.concatenate([oh, a], axis=1)
    acc = jnp.zeros((b, _EMB_DIM), jnp.float32)
    for tt in range(9):
        m = (t == float(tt)).astype(jnp.float32)[:, None]
        acc = acc + jnp.dot(phi * m, g_ref[tt],
                            preferred_element_type=jnp.float32)
    o_ref[:, :] = acc


def kernel(edge_attr, params):
    n = edge_attr.shape[0]
    g = _build_g(params)
    grid = n // _BLOCK
    return pl.pallas_call(
        _body,
        grid=(grid,),
        in_specs=[
            pl.BlockSpec((_BLOCK, 14), lambda i: (i, 0)),
            pl.BlockSpec((9, _PHI_W, _EMB_DIM), lambda i: (0, 0, 0)),
        ],
        out_specs=pl.BlockSpec((_BLOCK, _EMB_DIM), lambda i: (i, 0)),
        out_shape=jax.ShapeDtypeStruct((n, _EMB_DIM), jnp.float32),
    )(edge_attr, g)


# speedup vs baseline: 73.4941x; 2.4694x over previous
import numpy as np
import jax
import jax.numpy as jnp
from jax.experimental import pallas as pl

_EMB_DIM = 128
_EDGE_CONT = {0: [3, 6, 7, 9, 10, 11, 12, 13], 1: [2, 3, 4, 5, 6, 7, 9, 10, 11, 12, 13], 2: [2, 3, 4, 5, 6, 7, 9, 10, 11, 12, 13], 3: [1, 4, 5, 6, 7, 9, 10, 11, 12, 13], 4: [2, 3, 4, 5, 6, 7, 9, 10, 11, 12, 13], 5: [1, 2, 3, 4, 5, 6, 7, 9, 10, 11, 12, 13], 6: [2, 3, 4, 5, 6, 7, 9, 10, 11, 12, 13], 7: [1, 2, 3, 4, 5, 6, 7, 9, 10, 11, 12, 13], 8: [0, 1, 4, 6, 7, 9, 10, 11, 12, 13]}
_EDGE_DISC_FEATS = {0: [0, 1, 2, 4, 5, 8], 1: [0, 1, 8], 2: [0, 1, 8], 3: [0, 2, 3, 8], 4: [0, 1, 8], 5: [0, 8], 6: [0, 1, 8], 7: [0, 8], 8: [2, 3, 5, 8]}
_COL_RANGES = {0: 4, 1: 6, 2: 6, 3: 8, 4: 15, 5: 2, 8: 9}

_K = 256
_BLOCK = 3200

_DISC_LANES = []
for _c in [0, 1, 2, 3, 4, 5]:
    for _t in range(9):
        if _c in _EDGE_DISC_FEATS[_t]:
            for _v in range(_COL_RANGES[_c]):
                _DISC_LANES.append((_c, _t, _v))
for _v in range(9):
    _DISC_LANES.append((8, _v, _v))
_CONT_LANES = []
for _c in range(14):
    for _t in range(9):
        if _c in _EDGE_CONT[_t]:
            _CONT_LANES.append((_c, _t))
_ND = len(_DISC_LANES)
_NC = len(_CONT_LANES)
assert _ND + _NC <= _K

_SS = np.zeros((16, 2 * _K), np.float32)
for _L, (_c, _t, _v) in enumerate(_DISC_LANES):
    _SS[_c, _L] += 1.0
    _SS[8, _L] += 16.0
    _SS[14, _L] = -(_v + 16.0 * _t)
    _SS[14, _K + _L] = 1.0
for _i, (_c, _t) in enumerate(_CONT_LANES):
    _L = _ND + _i
    _SS[8, _L] = 16.0
    _SS[14, _L] = -16.0 * _t
    _SS[_c, _K + _L] = 1.0
_SS_BF = jnp.asarray(_SS, jnp.bfloat16)


def _col_spans(t):
    feats = _EDGE_DISC_FEATS[t]
    nd = len(feats)
    per, rem = _EMB_DIM // nd, _EMB_DIM % nd
    spans, col = {}, 0
    for i, f in enumerate(feats):
        dim = per + (1 if i < rem else 0)
        spans[f] = (col, dim)
        col += dim
    return spans


def _build_g(params):
    spans = {t: _col_spans(t) for t in range(9)}
    pieces = []
    i = 0
    while i < _ND:
        c, t, v = _DISC_LANES[i]
        if c != 8:
            r = _COL_RANGES[c]
            tbl = params["tables"][str(t)][str(c)][:r, :]
            lo, dim = spans[t][c]
            pieces.append(jnp.pad(tbl, ((0, 0), (lo, _EMB_DIM - lo - dim))))
            i += r
        else:
            lo, dim = spans[t][8]
            row = jnp.pad(params["tables"][str(t)]["8"][v:v + 1, :],
                          ((0, 0), (lo, _EMB_DIM - lo - dim)))
            pieces.append(row + params["b"][str(t)][None, :])
            i += 1
    for c, t in _CONT_LANES:
        pos = _EDGE_CONT[t].index(c)
        pieces.append(params["W"][str(t)][None, :, pos])
    pieces.append(jnp.zeros((_K - _ND - _NC, _EMB_DIM), jnp.float32))
    return jnp.concatenate(pieces, axis=0).astype(jnp.bfloat16)


def _body(a_ref, ss_ref, g_ref, o_ref):
    a = a_ref[:, :]
    b = a.shape[0]
    az = jnp.concatenate(
        [a, jnp.ones((b, 1), jnp.float32), jnp.zeros((b, 1), jnp.float32)],
        axis=1).astype(jnp.bfloat16)
    mm = jnp.dot(az, ss_ref[:, :], preferred_element_type=jnp.float32)
    phi = jnp.where(mm[:, :_K] == 0.0, mm[:, _K:], 0.0).astype(jnp.bfloat16)
    o_ref[:, :] = jnp.dot(phi, g_ref[:, :],
                          preferred_element_type=jnp.float32)


def kernel(edge_attr, params):
    n = edge_attr.shape[0]
    g = _build_g(params)
    grid = n // _BLOCK
    return pl.pallas_call(
        _body,
        grid=(grid,),
        in_specs=[
            pl.BlockSpec((_BLOCK, 14), lambda i: (i, 0)),
            pl.BlockSpec((16, 2 * _K), lambda i: (0, 0)),
            pl.BlockSpec((_K, _EMB_DIM), lambda i: (0, 0)),
        ],
        out_specs=pl.BlockSpec((_BLOCK, _EMB_DIM), lambda i: (i, 0)),
        out_shape=jax.ShapeDtypeStruct((n, _EMB_DIM), jnp.float32),
    )(edge_attr, _SS_BF, g)


# speedup vs baseline: 73.5634x; 1.0009x over previous
import numpy as np
import jax
import jax.numpy as jnp
from jax.experimental import pallas as pl

_EMB_DIM = 128
_EDGE_CONT = {0: [3, 6, 7, 9, 10, 11, 12, 13], 1: [2, 3, 4, 5, 6, 7, 9, 10, 11, 12, 13], 2: [2, 3, 4, 5, 6, 7, 9, 10, 11, 12, 13], 3: [1, 4, 5, 6, 7, 9, 10, 11, 12, 13], 4: [2, 3, 4, 5, 6, 7, 9, 10, 11, 12, 13], 5: [1, 2, 3, 4, 5, 6, 7, 9, 10, 11, 12, 13], 6: [2, 3, 4, 5, 6, 7, 9, 10, 11, 12, 13], 7: [1, 2, 3, 4, 5, 6, 7, 9, 10, 11, 12, 13], 8: [0, 1, 4, 6, 7, 9, 10, 11, 12, 13]}
_EDGE_DISC_FEATS = {0: [0, 1, 2, 4, 5, 8], 1: [0, 1, 8], 2: [0, 1, 8], 3: [0, 2, 3, 8], 4: [0, 1, 8], 5: [0, 8], 6: [0, 1, 8], 7: [0, 8], 8: [2, 3, 5, 8]}
_COL_RANGES = {0: 4, 1: 6, 2: 6, 3: 8, 4: 15, 5: 2, 8: 9}

_K = 256
_BLOCK = 3200

_DISC_LANES = []
for _c in [0, 1, 2, 3, 4, 5]:
    for _t in range(9):
        if _c in _EDGE_DISC_FEATS[_t]:
            for _v in range(_COL_RANGES[_c]):
                _DISC_LANES.append((_c, _t, _v))
for _v in range(9):
    _DISC_LANES.append((8, _v, _v))
_CONT_LANES = []
for _c in range(14):
    for _t in range(9):
        if _c in _EDGE_CONT[_t]:
            _CONT_LANES.append((_c, _t))
_ND = len(_DISC_LANES)
_NC = len(_CONT_LANES)
assert _ND + _NC <= _K

_SS = np.zeros((16, 2 * _K), np.float32)
for _L, (_c, _t, _v) in enumerate(_DISC_LANES):
    _SS[_c, _L] += 1.0
    _SS[8, _L] += 16.0
    _SS[14, _L] = -(_v + 16.0 * _t)
    _SS[14, _K + _L] = 1.0
for _i, (_c, _t) in enumerate(_CONT_LANES):
    _L = _ND + _i
    _SS[8, _L] = 16.0
    _SS[14, _L] = -16.0 * _t
    _SS[_c, _K + _L] = 1.0


def _col_spans(t):
    feats = _EDGE_DISC_FEATS[t]
    nd = len(feats)
    per, rem = _EMB_DIM // nd, _EMB_DIM % nd
    spans, col = {}, 0
    for i, f in enumerate(feats):
        dim = per + (1 if i < rem else 0)
        spans[f] = (col, dim)
        col += dim
    return spans


def _build_g(params):
    spans = {t: _col_spans(t) for t in range(9)}
    pieces = []
    i = 0
    while i < _ND:
        c, t, v = _DISC_LANES[i]
        if c != 8:
            r = _COL_RANGES[c]
            tbl = params["tables"][str(t)][str(c)][:r, :]
            lo, dim = spans[t][c]
            pieces.append(jnp.pad(tbl, ((0, 0), (lo, _EMB_DIM - lo - dim))))
            i += r
        else:
            lo, dim = spans[t][8]
            row = jnp.pad(params["tables"][str(t)]["8"][v:v + 1, :],
                          ((0, 0), (lo, _EMB_DIM - lo - dim)))
            pieces.append(row + params["b"][str(t)][None, :])
            i += 1
    for c, t in _CONT_LANES:
        pos = _EDGE_CONT[t].index(c)
        pieces.append(params["W"][str(t)][None, :, pos])
    pieces.append(jnp.zeros((_K - _ND - _NC, _EMB_DIM), jnp.float32))
    return jnp.concatenate(pieces, axis=0).astype(jnp.bfloat16)


def _body(a_ref, ss_ref, g_ref, o_ref):
    a = a_ref[:, :]
    b = a.shape[0]
    az = jnp.concatenate(
        [a, jnp.ones((b, 1), jnp.float32), jnp.zeros((b, 1), jnp.float32)],
        axis=1).astype(jnp.bfloat16)
    mm = jnp.dot(az, ss_ref[:, :], preferred_element_type=jnp.float32)
    phi = jnp.where(mm[:, :_K] == 0.0, mm[:, _K:], 0.0).astype(jnp.bfloat16)
    o_ref[:, :] = jnp.dot(phi, g_ref[:, :],
                          preferred_element_type=jnp.float32)


def kernel(edge_attr, params):
    n = edge_attr.shape[0]
    g = _build_g(params)
    grid = n // _BLOCK
    return pl.pallas_call(
        _body,
        grid=(grid,),
        in_specs=[
            pl.BlockSpec((_BLOCK, 14), lambda i: (i, 0)),
            pl.BlockSpec((16, 2 * _K), lambda i: (0, 0)),
            pl.BlockSpec((_K, _EMB_DIM), lambda i: (0, 0)),
        ],
        out_specs=pl.BlockSpec((_BLOCK, _EMB_DIM), lambda i: (i, 0)),
        out_shape=jax.ShapeDtypeStruct((n, _EMB_DIM), jnp.float32),
    )(edge_attr, jnp.asarray(_SS, jnp.bfloat16), g)
